# Initial kernel scaffold; baseline (speedup 1.0000x reference)
import functools

import jax
import jax.numpy as jnp
from jax import lax
from jax.experimental import pallas as pl
from jax.experimental.pallas import tpu as pltpu
from jax.experimental.pallas import tpu_sc as plsc

NC, NS, LANES = 2, 16, 16
NW = NC * NS
DEGW = 16
K = 128


def _sc_mesh():
    return plsc.VectorSubcoreMesh(
        core_axis_name="c", subcore_axis_name="s", num_cores=NC, num_subcores=NS
    )


def _deg_pass(dst, ones_kw, npad):
    e = dst.shape[0]
    ec = e // NW
    nfull, tail = divmod(ec, K)
    rows_per_tile = npad // NS

    @functools.partial(
        pl.kernel,
        out_type=jax.ShapeDtypeStruct((NC, npad, DEGW), jnp.float32),
        mesh=_sc_mesh(),
        scratch_types=[
            pltpu.VMEM_SHARED((npad, DEGW), jnp.float32),
            pltpu.VMEM((K, DEGW), jnp.float32),
            pltpu.VMEM((K,), jnp.int32),
            pltpu.VMEM((max(tail, 1),), jnp.int32),
        ],
    )
    def k(dst_hbm, ones_hbm, out_hbm, acc, ones_v, didx, didx_t):
        c = lax.axis_index("c")
        s = lax.axis_index("s")
        wid = s * NC + c
        pltpu.sync_copy(ones_hbm, ones_v)
        r0 = s * rows_per_tile
        for r in range(rows_per_tile // K):
            pltpu.sync_copy(ones_v, acc.at[pl.ds(r0 + r * K, K)])
        plsc.subcore_barrier()
        eb = wid * ec

        def chunk(i, carry):
            pltpu.sync_copy(dst_hbm.at[pl.ds(eb + i * K, K)], didx)
            pltpu.sync_copy(ones_v, acc.at[didx], add=True)
            return carry

        lax.fori_loop(0, nfull, chunk, 0)
        if tail:
            pltpu.sync_copy(dst_hbm.at[pl.ds(eb + nfull * K, tail)], didx_t)
            pltpu.sync_copy(ones_v.at[pl.ds(0, tail)], acc.at[didx_t], add=True)
        plsc.subcore_barrier()
        pltpu.sync_copy(
            acc.at[pl.ds(r0, rows_per_tile)],
            out_hbm.at[c, pl.ds(r0, rows_per_tile)],
        )

    return k(dst, ones_kw)


def _edge_pass(y, src, dst, npad):
    n, h = y.shape
    e = src.shape[0]
    ec = e // NW
    nfull, tail = divmod(ec, K)
    rows_per_tile = npad // NS
    init_rows = n // NS

    @functools.partial(
        pl.kernel,
        out_type=jax.ShapeDtypeStruct((NC, npad, h), jnp.float32),
        mesh=_sc_mesh(),
        scratch_types=[
            pltpu.VMEM_SHARED((npad, h), jnp.float32),
            pltpu.VMEM((K,), jnp.int32),
            pltpu.VMEM((K,), jnp.int32),
            pltpu.VMEM((K, h), jnp.float32),
            pltpu.VMEM((max(tail, 1),), jnp.int32),
            pltpu.VMEM((max(tail, 1),), jnp.int32),
            pltpu.VMEM((max(tail, 1), h), jnp.float32),
            pltpu.SemaphoreType.DMA,
        ],
    )
    def k(y_hbm, src_hbm, dst_hbm, out_hbm, acc, sidx, didx, rows,
          sidx_t, didx_t, rows_t, sem):
        c = lax.axis_index("c")
        s = lax.axis_index("s")
        wid = s * NC + c
        pltpu.sync_copy(
            y_hbm.at[pl.ds(s * init_rows, init_rows)],
            acc.at[pl.ds(s * init_rows, init_rows)],
        )
        plsc.subcore_barrier()
        eb = wid * ec

        def chunk(i, carry):
            b = eb + i * K
            pltpu.sync_copy(src_hbm.at[pl.ds(b, K)], sidx)
            pltpu.sync_copy(dst_hbm.at[pl.ds(b, K)], didx)
            pltpu.async_copy(y_hbm.at[sidx], rows, sem).wait()
            pltpu.sync_copy(rows, acc.at[didx], add=True)
            return carry

        lax.fori_loop(0, nfull, chunk, 0)
        if tail:
            b = eb + nfull * K
            pltpu.sync_copy(src_hbm.at[pl.ds(b, tail)], sidx_t)
            pltpu.sync_copy(dst_hbm.at[pl.ds(b, tail)], didx_t)
            pltpu.async_copy(y_hbm.at[sidx_t], rows_t, sem).wait()
            pltpu.sync_copy(rows_t, acc.at[didx_t], add=True)
        plsc.subcore_barrier()
        r0 = s * rows_per_tile
        pltpu.sync_copy(
            acc.at[pl.ds(r0, rows_per_tile)],
            out_hbm.at[c, pl.ds(r0, rows_per_tile)],
        )

    return k(y, src, dst)


def _tc1(x, w1, d0, d1, bn):
    n, d = x.shape
    h = w1.shape[1]
    grid = n // bn

    def body(x_ref, w_ref, d0_ref, d1_ref, y_ref, dinv_ref):
        deg = d0_ref[...] + d1_ref[...] - 1.0
        dinv = lax.rsqrt(deg)
        xw = jnp.dot(x_ref[...], w_ref[...], preferred_element_type=jnp.float32)
        y_ref[...] = xw * dinv
        dinv_ref[...] = dinv

    return pl.pallas_call(
        body,
        grid=(grid,),
        in_specs=[
            pl.BlockSpec((bn, d), lambda i: (i, 0)),
            pl.BlockSpec((d, h), lambda i: (0, 0)),
            pl.BlockSpec((bn, 1), lambda i: (i, 0)),
            pl.BlockSpec((bn, 1), lambda i: (i, 0)),
        ],
        out_specs=[
            pl.BlockSpec((bn, h), lambda i: (i, 0)),
            pl.BlockSpec((bn, 1), lambda i: (i, 0)),
        ],
        out_shape=[
            jax.ShapeDtypeStruct((n, h), jnp.float32),
            jax.ShapeDtypeStruct((n, 1), jnp.float32),
        ],
    )(x, w1, d0, d1)


def _tc2(a0, a1, y1, dinv, b1, w2, bn):
    n, h = y1.shape

    def body(a0_ref, a1_ref, y1_ref, dinv_ref, b1_ref, w2_ref, h_ref, y2_ref):
        conv = (a0_ref[...] + a1_ref[...] - y1_ref[...]) * dinv_ref[...] + b1_ref[...]
        hh = jnp.maximum(conv, 0.0)
        h_ref[...] = hh
        y2_ref[...] = (
            jnp.dot(hh, w2_ref[...], preferred_element_type=jnp.float32)
            * dinv_ref[...]
        )

    return pl.pallas_call(
        body,
        grid=(n // bn,),
        in_specs=[
            pl.BlockSpec((bn, h), lambda i: (i, 0)),
            pl.BlockSpec((bn, h), lambda i: (i, 0)),
            pl.BlockSpec((bn, h), lambda i: (i, 0)),
            pl.BlockSpec((bn, 1), lambda i: (i, 0)),
            pl.BlockSpec((1, h), lambda i: (0, 0)),
            pl.BlockSpec((h, h), lambda i: (0, 0)),
        ],
        out_specs=[
            pl.BlockSpec((bn, h), lambda i: (i, 0)),
            pl.BlockSpec((bn, h), lambda i: (i, 0)),
        ],
        out_shape=[
            jax.ShapeDtypeStruct((n, h), jnp.float32),
            jax.ShapeDtypeStruct((n, h), jnp.float32),
        ],
    )(a0, a1, y1, dinv, b1, w2)


def _tc3(hfeat, a0, a1, y2, dinv, b2, batch2d, wlin, blin, num_graphs, num_cls, bn):
    n, h = hfeat.shape
    grid = n // bn

    def body(h_ref, a0_ref, a1_ref, y2_ref, dinv_ref, b2_ref, bat_ref,
             wl_ref, bl_ref, out_ref, seg_scr, cnt_scr):
        i = pl.program_id(0)

        @pl.when(i == 0)
        def _init():
            seg_scr[...] = jnp.zeros_like(seg_scr)
            cnt_scr[...] = jnp.zeros_like(cnt_scr)

        conv = (a0_ref[...] + a1_ref[...] - y2_ref[...]) * dinv_ref[...] + b2_ref[...]
        h2 = jnp.maximum(conv, 0.0)
        sfeat = h_ref[...] + h2
        gids = lax.broadcasted_iota(jnp.int32, (1, num_graphs), 1)
        oh = (bat_ref[...] == gids).astype(jnp.float32)
        seg_scr[...] += lax.dot_general(
            oh, sfeat, (((0,), (0,)), ((), ())), preferred_element_type=jnp.float32
        )
        cnt_scr[...] += lax.dot_general(
            oh, jnp.ones((bn, 1), jnp.float32), (((0,), (0,)), ((), ())),
            preferred_element_type=jnp.float32,
        )

        @pl.when(i == grid - 1)
        def _fin():
            pool = seg_scr[...] / jnp.maximum(cnt_scr[...], 1.0)
            logits = (
                jnp.dot(pool, wl_ref[...], preferred_element_type=jnp.float32)
                + bl_ref[...]
            )
            logits = jnp.maximum(logits, 0.0)
            col = lax.broadcasted_iota(jnp.int32, logits.shape, 1)
            valid = col < num_cls
            masked = jnp.where(valid, logits, -jnp.inf)
            m = jnp.max(masked, axis=1, keepdims=True)
            z = logits - m
            ez = jnp.where(valid, jnp.exp(z), 0.0)
            se = jnp.sum(ez, axis=1, keepdims=True)
            out_ref[...] = jnp.where(valid, z - jnp.log(se), 0.0)

    return pl.pallas_call(
        body,
        grid=(grid,),
        in_specs=[
            pl.BlockSpec((bn, h), lambda i: (i, 0)),
            pl.BlockSpec((bn, h), lambda i: (i, 0)),
            pl.BlockSpec((bn, h), lambda i: (i, 0)),
            pl.BlockSpec((bn, h), lambda i: (i, 0)),
            pl.BlockSpec((bn, 1), lambda i: (i, 0)),
            pl.BlockSpec((1, h), lambda i: (0, 0)),
            pl.BlockSpec((bn, 1), lambda i: (i, 0)),
            pl.BlockSpec((h, h), lambda i: (0, 0)),
            pl.BlockSpec((1, h), lambda i: (0, 0)),
        ],
        out_specs=pl.BlockSpec((num_graphs, h), lambda i: (0, 0)),
        out_shape=jax.ShapeDtypeStruct((num_graphs, h), jnp.float32),
        scratch_shapes=[
            pltpu.VMEM((num_graphs, h), jnp.float32),
            pltpu.VMEM((num_graphs, 1), jnp.float32),
        ],
    )(hfeat, a0, a1, y2, dinv, b2, batch2d, wlin, blin)


def kernel(x, edge_index, batch, W1, b1, W2, b2, Wlin, blin):
    n, d = x.shape
    h = W1.shape[1]
    num_cls = Wlin.shape[1]
    num_graphs = 128
    src = edge_index[0]
    dst = edge_index[1]

    tile_quant = NS * K
    npad = ((n + tile_quant - 1) // tile_quant) * tile_quant

    bn = 400 if n % 400 == 0 else 200
    assert n % bn == 0 and n % NS == 0 and edge_index.shape[1] % NW == 0

    ones_kw = jnp.ones((K, DEGW), jnp.float32)
    degc = _deg_pass(dst, ones_kw, npad)
    d0 = degc[0, :n, :1]
    d1 = degc[1, :n, :1]

    y1, dinv = _tc1(x, W1, d0, d1, bn)
    acc1 = _edge_pass(y1, src, dst, npad)
    hfeat, y2 = _tc2(acc1[0, :n], acc1[1, :n], y1, dinv,
                     b1.reshape(1, -1), W2, bn)
    acc2 = _edge_pass(y2, src, dst, npad)

    wlin_pad = jnp.pad(Wlin, ((0, 0), (0, h - num_cls)))
    blin_pad = jnp.pad(blin, (0, h - num_cls)).reshape(1, -1)
    outp = _tc3(hfeat, acc2[0, :n], acc2[1, :n], y2, dinv,
                b2.reshape(1, -1), batch.reshape(-1, 1).astype(jnp.int32),
                wlin_pad, blin_pad, num_graphs, num_cls, bn)
    return outp[:, :num_cls]


# speedup vs baseline: 15.2758x; 15.2758x over previous
import functools

import jax
import jax.numpy as jnp
from jax import lax
from jax.experimental import pallas as pl
from jax.experimental.pallas import tpu as pltpu
from jax.experimental.pallas import tpu_sc as plsc

NC, NS, LANES = 2, 16, 16
NW = NC * NS
DEGW = 16
K = 128


def _sc_mesh():
    return plsc.VectorSubcoreMesh(
        core_axis_name="c", subcore_axis_name="s", num_cores=NC, num_subcores=NS
    )


def _deg_pass(dst, ones_kw, npad):
    e = dst.shape[0]
    ec = e // NW
    nfull, tail = divmod(ec, K)
    rows_per_tile = npad // NS

    @functools.partial(
        pl.kernel,
        out_type=jax.ShapeDtypeStruct((NC, npad, DEGW), jnp.float32),
        mesh=_sc_mesh(),
        scratch_types=[
            pltpu.VMEM_SHARED((npad, DEGW), jnp.float32),
            pltpu.VMEM((K, DEGW), jnp.float32),
            pltpu.VMEM((K,), jnp.int32),
            pltpu.VMEM((max(tail, 1),), jnp.int32),
        ],
    )
    def k(dst_hbm, ones_hbm, out_hbm, acc, ones_v, didx, didx_t):
        c = lax.axis_index("c")
        s = lax.axis_index("s")
        wid = s * NC + c
        pltpu.sync_copy(ones_hbm, ones_v)
        r0 = s * rows_per_tile
        for r in range(rows_per_tile // K):
            pltpu.sync_copy(ones_v, acc.at[pl.ds(r0 + r * K, K)])
        plsc.subcore_barrier()
        eb = wid * ec

        def chunk(i, carry):
            pltpu.sync_copy(dst_hbm.at[pl.ds(eb + i * K, K)], didx)
            pltpu.sync_copy(ones_v, acc.at[didx], add=True)
            return carry

        lax.fori_loop(0, nfull, chunk, 0)
        if tail:
            pltpu.sync_copy(dst_hbm.at[pl.ds(eb + nfull * K, tail)], didx_t)
            pltpu.sync_copy(ones_v.at[pl.ds(0, tail)], acc.at[didx_t], add=True)
        plsc.subcore_barrier()
        pltpu.sync_copy(
            acc.at[pl.ds(r0, rows_per_tile)],
            out_hbm.at[c, pl.ds(r0, rows_per_tile)],
        )

    return k(dst, ones_kw)


def _edge_pass(y, src, dst, npad):
    n, h = y.shape
    assert n == npad
    e = src.shape[0]
    ec = e // NW
    nfull, tail = divmod(ec, K)
    rows_per_tile = npad // NS

    @functools.partial(
        pl.kernel,
        out_type=jax.ShapeDtypeStruct((NC, npad, h), jnp.float32),
        mesh=_sc_mesh(),
        scratch_types=[
            pltpu.VMEM_SHARED((npad, h), jnp.float32),
            pltpu.VMEM((K,), jnp.int32),
            pltpu.VMEM((K,), jnp.int32),
            pltpu.VMEM((K, h), jnp.float32),
            pltpu.VMEM((max(tail, 1),), jnp.int32),
            pltpu.VMEM((max(tail, 1),), jnp.int32),
            pltpu.VMEM((max(tail, 1), h), jnp.float32),
            pltpu.SemaphoreType.DMA,
        ],
    )
    def k(y_hbm, src_hbm, dst_hbm, out_hbm, acc, sidx, didx, rows,
          sidx_t, didx_t, rows_t, sem):
        c = lax.axis_index("c")
        s = lax.axis_index("s")
        wid = s * NC + c
        pltpu.sync_copy(
            y_hbm.at[pl.ds(s * rows_per_tile, rows_per_tile)],
            acc.at[pl.ds(s * rows_per_tile, rows_per_tile)],
        )
        plsc.subcore_barrier()
        eb = wid * ec

        def chunk(i, carry):
            b = eb + i * K
            pltpu.sync_copy(src_hbm.at[pl.ds(b, K)], sidx)
            pltpu.sync_copy(dst_hbm.at[pl.ds(b, K)], didx)
            pltpu.async_copy(y_hbm.at[sidx], rows, sem).wait()
            pltpu.sync_copy(rows, acc.at[didx], add=True)
            return carry

        lax.fori_loop(0, nfull, chunk, 0)
        if tail:
            b = eb + nfull * K
            pltpu.sync_copy(src_hbm.at[pl.ds(b, tail)], sidx_t)
            pltpu.sync_copy(dst_hbm.at[pl.ds(b, tail)], didx_t)
            pltpu.async_copy(y_hbm.at[sidx_t], rows_t, sem).wait()
            pltpu.sync_copy(rows_t, acc.at[didx_t], add=True)
        plsc.subcore_barrier()
        r0 = s * rows_per_tile
        pltpu.sync_copy(
            acc.at[pl.ds(r0, rows_per_tile)],
            out_hbm.at[c, pl.ds(r0, rows_per_tile)],
        )

    return k(y, src, dst)


def _tc1(x, w1, d0, d1, bn):
    n, d = x.shape
    h = w1.shape[1]
    grid = n // bn

    def body(x_ref, w_ref, d0_ref, d1_ref, y_ref, dinv_ref):
        deg = d0_ref[...] + d1_ref[...] - 1.0
        dinv = lax.rsqrt(deg)
        xw = jnp.dot(x_ref[...], w_ref[...], preferred_element_type=jnp.float32)
        y_ref[...] = xw * dinv
        dinv_ref[...] = dinv

    return pl.pallas_call(
        body,
        grid=(grid,),
        in_specs=[
            pl.BlockSpec((bn, d), lambda i: (i, 0)),
            pl.BlockSpec((d, h), lambda i: (0, 0)),
            pl.BlockSpec((bn, 1), lambda i: (i, 0)),
            pl.BlockSpec((bn, 1), lambda i: (i, 0)),
        ],
        out_specs=[
            pl.BlockSpec((bn, h), lambda i: (i, 0)),
            pl.BlockSpec((bn, 1), lambda i: (i, 0)),
        ],
        out_shape=[
            jax.ShapeDtypeStruct((n, h), jnp.float32),
            jax.ShapeDtypeStruct((n, 1), jnp.float32),
        ],
    )(x, w1, d0, d1)


def _tc2(a0, a1, y1, dinv, b1, w2, bn):
    n, h = y1.shape

    def body(a0_ref, a1_ref, y1_ref, dinv_ref, b1_ref, w2_ref, h_ref, y2_ref):
        conv = (a0_ref[...] + a1_ref[...] - y1_ref[...]) * dinv_ref[...] + b1_ref[...]
        hh = jnp.maximum(conv, 0.0)
        h_ref[...] = hh
        y2_ref[...] = (
            jnp.dot(hh, w2_ref[...], preferred_element_type=jnp.float32)
            * dinv_ref[...]
        )

    return pl.pallas_call(
        body,
        grid=(n // bn,),
        in_specs=[
            pl.BlockSpec((bn, h), lambda i: (i, 0)),
            pl.BlockSpec((bn, h), lambda i: (i, 0)),
            pl.BlockSpec((bn, h), lambda i: (i, 0)),
            pl.BlockSpec((bn, 1), lambda i: (i, 0)),
            pl.BlockSpec((1, h), lambda i: (0, 0)),
            pl.BlockSpec((h, h), lambda i: (0, 0)),
        ],
        out_specs=[
            pl.BlockSpec((bn, h), lambda i: (i, 0)),
            pl.BlockSpec((bn, h), lambda i: (i, 0)),
        ],
        out_shape=[
            jax.ShapeDtypeStruct((n, h), jnp.float32),
            jax.ShapeDtypeStruct((n, h), jnp.float32),
        ],
    )(a0, a1, y1, dinv, b1, w2)


def _tc3(hfeat, a0, a1, y2, dinv, b2, batch2d, wlin, blin, num_graphs, num_cls, bn):
    n, h = hfeat.shape
    grid = n // bn

    def body(h_ref, a0_ref, a1_ref, y2_ref, dinv_ref, b2_ref, bat_ref,
             wl_ref, bl_ref, out_ref, seg_scr, cnt_scr):
        i = pl.program_id(0)

        @pl.when(i == 0)
        def _init():
            seg_scr[...] = jnp.zeros_like(seg_scr)
            cnt_scr[...] = jnp.zeros_like(cnt_scr)

        conv = (a0_ref[...] + a1_ref[...] - y2_ref[...]) * dinv_ref[...] + b2_ref[...]
        h2 = jnp.maximum(conv, 0.0)
        sfeat = h_ref[...] + h2
        gids = lax.broadcasted_iota(jnp.int32, (1, num_graphs), 1)
        oh = (bat_ref[...] == gids).astype(jnp.float32)
        seg_scr[...] += lax.dot_general(
            oh, sfeat, (((0,), (0,)), ((), ())), preferred_element_type=jnp.float32
        )
        cnt_scr[...] += lax.dot_general(
            oh, jnp.ones((bn, 1), jnp.float32), (((0,), (0,)), ((), ())),
            preferred_element_type=jnp.float32,
        )

        @pl.when(i == grid - 1)
        def _fin():
            pool = seg_scr[...] / jnp.maximum(cnt_scr[...], 1.0)
            logits = (
                jnp.dot(pool, wl_ref[...], preferred_element_type=jnp.float32)
                + bl_ref[...]
            )
            logits = jnp.maximum(logits, 0.0)
            col = lax.broadcasted_iota(jnp.int32, logits.shape, 1)
            valid = col < num_cls
            masked = jnp.where(valid, logits, -jnp.inf)
            m = jnp.max(masked, axis=1, keepdims=True)
            z = logits - m
            ez = jnp.where(valid, jnp.exp(z), 0.0)
            se = jnp.sum(ez, axis=1, keepdims=True)
            out_ref[...] = jnp.where(valid, z - jnp.log(se), 0.0)

    return pl.pallas_call(
        body,
        grid=(grid,),
        in_specs=[
            pl.BlockSpec((bn, h), lambda i: (i, 0)),
            pl.BlockSpec((bn, h), lambda i: (i, 0)),
            pl.BlockSpec((bn, h), lambda i: (i, 0)),
            pl.BlockSpec((bn, h), lambda i: (i, 0)),
            pl.BlockSpec((bn, 1), lambda i: (i, 0)),
            pl.BlockSpec((1, h), lambda i: (0, 0)),
            pl.BlockSpec((bn, 1), lambda i: (i, 0)),
            pl.BlockSpec((h, h), lambda i: (0, 0)),
            pl.BlockSpec((1, h), lambda i: (0, 0)),
        ],
        out_specs=pl.BlockSpec((num_graphs, h), lambda i: (0, 0)),
        out_shape=jax.ShapeDtypeStruct((num_graphs, h), jnp.float32),
        scratch_shapes=[
            pltpu.VMEM((num_graphs, h), jnp.float32),
            pltpu.VMEM((num_graphs, 1), jnp.float32),
        ],
    )(hfeat, a0, a1, y2, dinv, b2, batch2d, wlin, blin)


def kernel(x, edge_index, batch, W1, b1, W2, b2, Wlin, blin):
    n, d = x.shape
    h = W1.shape[1]
    num_cls = Wlin.shape[1]
    num_graphs = 128
    src = edge_index[0]
    dst = edge_index[1]

    tile_quant = NS * K
    npad = ((n + tile_quant - 1) // tile_quant) * tile_quant
    bn = 512
    assert edge_index.shape[1] % NW == 0

    xp = jnp.pad(x, ((0, npad - n), (0, 0)))
    batchp = jnp.pad(batch.astype(jnp.int32), (0, npad - n),
                     constant_values=num_graphs)

    ones_kw = jnp.ones((K, DEGW), jnp.float32)
    degc = _deg_pass(dst, ones_kw, npad)
    d0 = degc[0, :, :1]
    d1 = degc[1, :, :1]

    y1, dinv = _tc1(xp, W1, d0, d1, bn)
    acc1 = _edge_pass(y1, src, dst, npad)
    hfeat, y2 = _tc2(acc1[0], acc1[1], y1, dinv,
                     b1.reshape(1, -1), W2, bn)
    acc2 = _edge_pass(y2, src, dst, npad)

    wlin_pad = jnp.pad(Wlin, ((0, 0), (0, h - num_cls)))
    blin_pad = jnp.pad(blin, (0, h - num_cls)).reshape(1, -1)
    outp = _tc3(hfeat, acc2[0], acc2[1], y2, dinv,
                b2.reshape(1, -1), batchp.reshape(-1, 1),
                wlin_pad, blin_pad, num_graphs, num_cls, bn)
    return outp[:, :num_cls]
